# baseline (speedup 1.0000x reference)
import functools

import jax
import jax.numpy as jnp
from jax import lax
from jax.experimental import pallas as pl
from jax.experimental.pallas import tpu as pltpu
from jax.experimental.pallas import tpu_sc as plsc

F32 = jnp.float32
NC = 2
NS = 16
LANES = 16
H = 64


def _silu(x):
    xc = jnp.maximum(x, -30.0)
    d = 1.0 + jnp.exp(-xc)
    y = 1.0 / d
    y = y * (2.0 - d * y)
    return xc * y


def _tc_ab(x, w, b, block_rows=1000):
    n = x.shape[0]

    def body(x_ref, w_ref, b_ref, a_ref, b_out_ref):
        ab = jnp.dot(x_ref[...], w_ref[...], preferred_element_type=F32)
        ab = ab + b_ref[...]
        a_ref[...] = ab[:, :H]
        b_out_ref[...] = ab[:, H:]

    grid = (n // block_rows,)
    return pl.pallas_call(
        body,
        grid=grid,
        in_specs=[
            pl.BlockSpec((block_rows, x.shape[1]), lambda i: (i, 0)),
            pl.BlockSpec(w.shape, lambda i: (0, 0)),
            pl.BlockSpec((1, 2 * H), lambda i: (0, 0)),
        ],
        out_specs=[
            pl.BlockSpec((block_rows, H), lambda i: (i, 0)),
            pl.BlockSpec((block_rows, H), lambda i: (i, 0)),
        ],
        out_shape=[
            jax.ShapeDtypeStruct((n, H), F32),
            jax.ShapeDtypeStruct((n, H), F32),
        ],
    )(x, w, b.reshape(1, 2 * H))


def _tc_update(h, agg0, agg1, wut, wub, bu, wn, bn, block_rows=1000):
    n = h.shape[0]

    def body(h_ref, a0_ref, a1_ref, wut_ref, wub_ref, bu_ref, wn_ref, bn_ref,
             hn_ref, a_ref, b_ref):
        agg = a0_ref[...] + a1_ref[...]
        hu = (jnp.dot(h_ref[...], wut_ref[...], preferred_element_type=F32)
              + jnp.dot(agg, wub_ref[...], preferred_element_type=F32)
              + bu_ref[...])
        hn = jax.nn.silu(hu)
        hn_ref[...] = hn
        ab = jnp.dot(hn, wn_ref[...], preferred_element_type=F32) + bn_ref[...]
        a_ref[...] = ab[:, :H]
        b_ref[...] = ab[:, H:]

    grid = (n // block_rows,)
    bs_rows = lambda w: pl.BlockSpec((block_rows, w), lambda i: (i, 0))
    bs_full = lambda a: pl.BlockSpec(a.shape, lambda i: (0, 0))
    return pl.pallas_call(
        body,
        grid=grid,
        in_specs=[
            bs_rows(H), bs_rows(H), bs_rows(H),
            bs_full(wut), bs_full(wub),
            pl.BlockSpec((1, H), lambda i: (0, 0)),
            bs_full(wn),
            pl.BlockSpec((1, 2 * H), lambda i: (0, 0)),
        ],
        out_specs=[bs_rows(H), bs_rows(H), bs_rows(H)],
        out_shape=[
            jax.ShapeDtypeStruct((n, H), F32),
            jax.ShapeDtypeStruct((n, H), F32),
            jax.ShapeDtypeStruct((n, H), F32),
        ],
    )(h, agg0, agg1, wut, wub, bu.reshape(1, H), wn, bn.reshape(1, 2 * H))


def _sc_edge(a_tab, b_tab, src3d, dst3d):
    n = a_tab.shape[0]
    _, nch, c_sz = src3d.shape
    rpt = n // NS

    mesh = plsc.VectorSubcoreMesh(core_axis_name="c", subcore_axis_name="s")

    @functools.partial(
        pl.kernel,
        out_type=(jax.ShapeDtypeStruct((NS, rpt, H), F32),
                  jax.ShapeDtypeStruct((NS, rpt, H), F32)),
        mesh=mesh,
        compiler_params=pltpu.CompilerParams(use_tc_tiling_on_sc=False),
        scratch_types=[
            pltpu.VMEM((nch, c_sz), jnp.int32),
            pltpu.VMEM((nch, c_sz), jnp.int32),
            pltpu.VMEM((c_sz, H), F32),
            pltpu.VMEM((c_sz, H), F32),
            pltpu.VMEM((c_sz, H), F32),
            pltpu.VMEM((rpt // 5, H), F32),
            pltpu.VMEM_SHARED((n, H), F32),
            pltpu.SemaphoreType.DMA,
            pltpu.SemaphoreType.DMA,
        ],
    )
    def k(a_hbm, b_hbm, src_hbm, dst_hbm, out0, out1,
          srcv, dstv, av, bv, mv, stage, aggsh, sem1, sem2):
        c = lax.axis_index("c")
        s = lax.axis_index("s")
        g = c * NS + s

        q_sz = rpt // 5
        def zrow(i, _):
            for j in range(H // LANES):
                stage[i, pl.ds(j * LANES, LANES)] = jnp.zeros((LANES,), F32)
            return 0
        lax.fori_loop(0, q_sz, zrow, 0)
        def zq(q, _):
            pltpu.sync_copy(stage, aggsh.at[pl.ds(s * rpt + q * q_sz, q_sz)])
            return 0
        lax.fori_loop(0, 5, zq, 0)

        pltpu.sync_copy(src_hbm.at[g], srcv)
        pltpu.sync_copy(dst_hbm.at[g], dstv)
        plsc.subcore_barrier()

        def chunk(kk, _):
            cp1 = pltpu.async_copy(a_hbm.at[srcv.at[kk]], av, sem1)
            cp2 = pltpu.async_copy(b_hbm.at[dstv.at[kk]], bv, sem2)
            cp1.wait()
            cp2.wait()

            def ebody(i, _):
                for j in range(H // LANES):
                    sl = pl.ds(j * LANES, LANES)
                    x = av[i, sl] + bv[i, sl]
                    mv[i, sl] = _silu(x)
                return 0
            lax.fori_loop(0, c_sz, ebody, 0)
            pltpu.sync_copy(mv, aggsh.at[dstv.at[kk]], add=True)
            return 0
        lax.fori_loop(0, nch, chunk, 0)

        plsc.subcore_barrier()

        def cq(q, _):
            pltpu.sync_copy(aggsh.at[pl.ds(s * rpt + q * q_sz, q_sz)], stage)

            @pl.when(c == 0)
            def _():
                pltpu.sync_copy(stage, out0.at[s, pl.ds(q * q_sz, q_sz)])

            @pl.when(c == 1)
            def _():
                pltpu.sync_copy(stage, out1.at[s, pl.ds(q * q_sz, q_sz)])
            return 0
        lax.fori_loop(0, 5, cq, 0)

    return k(a_tab, b_tab, src3d, dst3d)


def _sc_edge_final(p_tab, q_tab, src3d, dst3d):
    _, nch, c_sz = src3d.shape
    ept = nch * c_sz
    e = NC * NS * ept

    mesh = plsc.VectorSubcoreMesh(core_axis_name="c", subcore_axis_name="s")

    @functools.partial(
        pl.kernel,
        out_type=jax.ShapeDtypeStruct((e, H), F32),
        mesh=mesh,
        compiler_params=pltpu.CompilerParams(use_tc_tiling_on_sc=False),
        scratch_types=[
            pltpu.VMEM((nch, c_sz), jnp.int32),
            pltpu.VMEM((nch, c_sz), jnp.int32),
            pltpu.VMEM((c_sz, H), F32),
            pltpu.VMEM((c_sz, H), F32),
            pltpu.VMEM((c_sz, H), F32),
            pltpu.SemaphoreType.DMA,
            pltpu.SemaphoreType.DMA,
        ],
    )
    def k(p_hbm, q_hbm, src_hbm, dst_hbm, uout,
          srcv, dstv, av, bv, uv, sem1, sem2):
        c = lax.axis_index("c")
        s = lax.axis_index("s")
        g = c * NS + s

        pltpu.sync_copy(src_hbm.at[g], srcv)
        pltpu.sync_copy(dst_hbm.at[g], dstv)

        def chunk(kk, _):
            cp1 = pltpu.async_copy(p_hbm.at[srcv.at[kk]], av, sem1)
            cp2 = pltpu.async_copy(q_hbm.at[dstv.at[kk]], bv, sem2)
            cp1.wait()
            cp2.wait()

            def ebody(i, _):
                for j in range(H // LANES):
                    sl = pl.ds(j * LANES, LANES)
                    x = av[i, sl] + bv[i, sl]
                    uv[i, sl] = _silu(x)
                return 0
            lax.fori_loop(0, c_sz, ebody, 0)
            pltpu.sync_copy(uv, uout.at[pl.ds(g * ept + kk * c_sz, c_sz)])
            return 0
        lax.fori_loop(0, nch, chunk, 0)

    return k(p_tab, q_tab, src3d, dst3d)


def _tc_rate(u, w2, b2, block_rows=8000):
    e = u.shape[0]

    def body(u_ref, w2_ref, b2_ref, r_ref):
        z = jnp.dot(u_ref[...], w2_ref[...], preferred_element_type=F32)
        r_ref[...] = jax.nn.softplus(z + b2_ref[...])

    return pl.pallas_call(
        body,
        grid=(e // block_rows,),
        in_specs=[
            pl.BlockSpec((block_rows, H), lambda i: (i, 0)),
            pl.BlockSpec((H, 1), lambda i: (0, 0)),
            pl.BlockSpec((1, 1), lambda i: (0, 0)),
        ],
        out_specs=pl.BlockSpec((block_rows, 1), lambda i: (i, 0)),
        out_shape=jax.ShapeDtypeStruct((e, 1), F32),
    )(u, w2, b2.reshape(1, 1))


def _sc_matrix(src4d, dst4d, r4d, n):
    _, nhalf, half, c_sz = src4d.shape
    nn = n * n
    zpt = nn // (NC * NS)
    zbuf = 25000
    nzcp = zpt // zbuf

    ZU = 5
    mesh = plsc.VectorSubcoreMesh(core_axis_name="c", subcore_axis_name="s")

    @functools.partial(
        pl.kernel,
        out_type=jax.ShapeDtypeStruct((nn,), F32),
        mesh=mesh,
        compiler_params=pltpu.CompilerParams(use_tc_tiling_on_sc=False),
        scratch_types=[
            pltpu.VMEM((half, c_sz), jnp.int32),
            pltpu.VMEM((half, c_sz), jnp.int32),
            pltpu.VMEM((half, c_sz), F32),
            pltpu.VMEM((2, c_sz), jnp.int32),
            pltpu.VMEM((zbuf,), F32),
            pltpu.SemaphoreType.DMA,
            pltpu.SemaphoreType.DMA,
            pltpu.SemaphoreType.DMA,
            pltpu.SemaphoreType.DMA,
            pltpu.SemaphoreType.DMA,
        ],
    )
    def k(src_hbm, dst_hbm, r_hbm, m_out, srcv, dstv, rv, idxv, zv,
          z0, z1, z2, z3, z4):
        c = lax.axis_index("c")
        s = lax.axis_index("s")
        g = c * NS + s

        def zfill(i, _):
            zv[pl.ds(i * LANES, LANES)] = jnp.zeros((LANES,), F32)
            return 0
        lax.fori_loop(0, zbuf // LANES, zfill, 0)
        if zbuf % LANES:
            zv[pl.ds(zbuf - LANES, LANES)] = jnp.zeros((LANES,), F32)

        zsems = (z0, z1, z2, z3, z4)

        def zout(i, _):
            base = g * zpt + i * (ZU * zbuf)
            cps = []
            for u in range(ZU):
                off = pl.multiple_of(base + u * zbuf, 8)
                cps.append(pltpu.async_copy(
                    zv, m_out.at[pl.ds(off, zbuf)], zsems[u]))
            for cp in cps:
                cp.wait()
            return 0
        lax.fori_loop(0, nzcp // ZU, zout, 0)
        plsc.subcore_barrier()

        for h in range(nhalf):
            pltpu.sync_copy(src_hbm.at[s, h], srcv)
            pltpu.sync_copy(dst_hbm.at[s, h], dstv)
            pltpu.sync_copy(r_hbm.at[s, h], rv)

            def chunk2(kk, _):
                def ib0(j, _):
                    sl = pl.ds(j * LANES, LANES)
                    idxv[0, sl] = srcv[2 * kk, sl] * n + dstv[2 * kk, sl]
                    return 0
                lax.fori_loop(0, c_sz // LANES, ib0, 0)
                cp0 = pltpu.async_copy(rv.at[2 * kk], m_out.at[idxv.at[0]], z0)

                def ib1(j, _):
                    sl = pl.ds(j * LANES, LANES)
                    idxv[1, sl] = (srcv[2 * kk + 1, sl] * n
                                   + dstv[2 * kk + 1, sl])
                    return 0
                lax.fori_loop(0, c_sz // LANES, ib1, 0)
                cp1 = pltpu.async_copy(rv.at[2 * kk + 1],
                                       m_out.at[idxv.at[1]], z1)
                cp0.wait()
                cp1.wait()
                return 0
            lax.fori_loop(0, half // 2, chunk2, 0)

    return k(src4d, dst4d, r4d)


def _tc_diag(m, block_rows=128):
    n = m.shape[0]

    def body(m_ref, tile_ref, out_ref):
        rowsum = jnp.sum(m_ref[...], axis=1)
        rr = lax.broadcasted_iota(jnp.int32, (block_rows, block_rows), 0)
        cc = lax.broadcasted_iota(jnp.int32, (block_rows, block_rows), 1)
        out_ref[...] = jnp.where(rr == cc, -rowsum[:, None], tile_ref[...])

    return pl.pallas_call(
        body,
        grid=(pl.cdiv(n, block_rows),),
        in_specs=[
            pl.BlockSpec((block_rows, n), lambda i: (i, 0)),
            pl.BlockSpec((block_rows, block_rows), lambda i: (i, i)),
        ],
        out_specs=pl.BlockSpec((block_rows, block_rows), lambda i: (i, i)),
        out_shape=jax.ShapeDtypeStruct((n, n), F32),
        input_output_aliases={0: 0},
    )(m, m)


def kernel(mu, t, context, edge_index, mp_params, edge_mlp_params):
    n = mu.shape[0]
    e = edge_index.shape[1]
    c_sz = 80

    src = edge_index[0].astype(jnp.int32)
    dst = edge_index[1].astype(jnp.int32)
    nw = NC * NS
    nch = e // (nw * c_sz)
    src3d = src.reshape(nw, nch, c_sz)
    dst3d = dst.reshape(nw, nch, c_sz)
    half9 = e // (NS * 2 * c_sz)
    src4d = src.reshape(NS, 2, half9, c_sz)
    dst4d = dst.reshape(NS, 2, half9, c_sz)

    ctx = context.astype(F32)
    d0 = 2 + ctx.shape[1]
    h0 = jnp.concatenate(
        [mu.reshape(n, 1), jnp.broadcast_to(t.reshape(1, 1), (n, 1)), ctx,
         jnp.zeros((n, H - d0), F32)], axis=1)

    def pad_rows(w):
        return jnp.concatenate(
            [w, jnp.zeros((H - w.shape[0], w.shape[1]), F32)], axis=0)

    w1, b1, w2, b2 = edge_mlp_params
    msg_w, msg_b, upd_wt, upd_wb, upd_b = [], [], [], [], []
    for (wm, bm, wu, bu) in mp_params:
        d = wm.shape[0] // 2
        msg_w.append(jnp.concatenate(
            [pad_rows(wm[:d]), pad_rows(wm[d:])], axis=1))
        msg_b.append(jnp.concatenate([bm, jnp.zeros((H,), F32)]))
        upd_wt.append(pad_rows(wu[:d]))
        upd_wb.append(wu[d:])
        upd_b.append(bu)
    msg_w.append(jnp.concatenate([w1[:H], w1[H:]], axis=1))
    msg_b.append(jnp.concatenate([b1, jnp.zeros((H,), F32)]))

    a_tab, b_tab = _tc_ab(h0, msg_w[0], msg_b[0])
    h = h0
    for l in range(len(mp_params)):
        agg0, agg1 = _sc_edge(a_tab, b_tab, src3d, dst3d)
        agg0 = agg0.reshape(n, H)
        agg1 = agg1.reshape(n, H)
        h, a_tab, b_tab = _tc_update(
            h, agg0, agg1, upd_wt[l], upd_wb[l], upd_b[l],
            msg_w[l + 1], msg_b[l + 1])

    u = _sc_edge_final(a_tab, b_tab, src3d, dst3d)
    r = _tc_rate(u, w2, b2).reshape(e)
    m_flat = _sc_matrix(src4d, dst4d, r.reshape(NS, 2, half9, c_sz), n)
    return _tc_diag(m_flat.reshape(n, n))


# speedup vs baseline: 1.0765x; 1.0765x over previous
import functools

import jax
import jax.numpy as jnp
from jax import lax
from jax.experimental import pallas as pl
from jax.experimental.pallas import tpu as pltpu
from jax.experimental.pallas import tpu_sc as plsc

F32 = jnp.float32
NC = 2
NS = 16
LANES = 16
H = 64


def _silu(x):
    xc = jnp.maximum(x, -30.0)
    d = 1.0 + jnp.exp(-xc)
    y = 1.0 / d
    y = y * (2.0 - d * y)
    return xc * y


def _tc_ab(x, w, b, block_rows=1000):
    n = x.shape[0]

    def body(x_ref, w_ref, b_ref, a_ref, b_out_ref):
        ab = jnp.dot(x_ref[...], w_ref[...], preferred_element_type=F32)
        ab = ab + b_ref[...]
        a_ref[...] = ab[:, :H]
        b_out_ref[...] = ab[:, H:]

    grid = (n // block_rows,)
    return pl.pallas_call(
        body,
        grid=grid,
        in_specs=[
            pl.BlockSpec((block_rows, x.shape[1]), lambda i: (i, 0)),
            pl.BlockSpec(w.shape, lambda i: (0, 0)),
            pl.BlockSpec((1, 2 * H), lambda i: (0, 0)),
        ],
        out_specs=[
            pl.BlockSpec((block_rows, H), lambda i: (i, 0)),
            pl.BlockSpec((block_rows, H), lambda i: (i, 0)),
        ],
        out_shape=[
            jax.ShapeDtypeStruct((n, H), F32),
            jax.ShapeDtypeStruct((n, H), F32),
        ],
    )(x, w, b.reshape(1, 2 * H))


def _tc_update(h, agg0, agg1, wut, wub, bu, wn, bn, block_rows=1000):
    n = h.shape[0]

    def body(h_ref, a0_ref, a1_ref, wut_ref, wub_ref, bu_ref, wn_ref, bn_ref,
             hn_ref, a_ref, b_ref):
        agg = a0_ref[...] + a1_ref[...]
        hu = (jnp.dot(h_ref[...], wut_ref[...], preferred_element_type=F32)
              + jnp.dot(agg, wub_ref[...], preferred_element_type=F32)
              + bu_ref[...])
        hn = jax.nn.silu(hu)
        hn_ref[...] = hn
        ab = jnp.dot(hn, wn_ref[...], preferred_element_type=F32) + bn_ref[...]
        a_ref[...] = ab[:, :H]
        b_ref[...] = ab[:, H:]

    grid = (n // block_rows,)
    bs_rows = lambda w: pl.BlockSpec((block_rows, w), lambda i: (i, 0))
    bs_full = lambda a: pl.BlockSpec(a.shape, lambda i: (0, 0))
    return pl.pallas_call(
        body,
        grid=grid,
        in_specs=[
            bs_rows(H), bs_rows(H), bs_rows(H),
            bs_full(wut), bs_full(wub),
            pl.BlockSpec((1, H), lambda i: (0, 0)),
            bs_full(wn),
            pl.BlockSpec((1, 2 * H), lambda i: (0, 0)),
        ],
        out_specs=[bs_rows(H), bs_rows(H), bs_rows(H)],
        out_shape=[
            jax.ShapeDtypeStruct((n, H), F32),
            jax.ShapeDtypeStruct((n, H), F32),
            jax.ShapeDtypeStruct((n, H), F32),
        ],
    )(h, agg0, agg1, wut, wub, bu.reshape(1, H), wn, bn.reshape(1, 2 * H))


def _sc_edge(a_tab, b_tab, src3d, dst3d):
    n = a_tab.shape[0]
    _, nch, c_sz = src3d.shape
    rpt = n // NS

    mesh = plsc.VectorSubcoreMesh(core_axis_name="c", subcore_axis_name="s")

    @functools.partial(
        pl.kernel,
        out_type=(jax.ShapeDtypeStruct((NS, rpt, H), F32),
                  jax.ShapeDtypeStruct((NS, rpt, H), F32)),
        mesh=mesh,
        compiler_params=pltpu.CompilerParams(use_tc_tiling_on_sc=False),
        scratch_types=[
            pltpu.VMEM((nch, c_sz), jnp.int32),
            pltpu.VMEM((nch, c_sz), jnp.int32),
            pltpu.VMEM((c_sz, H), F32),
            pltpu.VMEM((c_sz, H), F32),
            pltpu.VMEM((c_sz, H), F32),
            pltpu.VMEM((rpt // 5, H), F32),
            pltpu.VMEM_SHARED((n, H), F32),
            pltpu.SemaphoreType.DMA,
            pltpu.SemaphoreType.DMA,
        ],
    )
    def k(a_hbm, b_hbm, src_hbm, dst_hbm, out0, out1,
          srcv, dstv, av, bv, mv, stage, aggsh, sem1, sem2):
        c = lax.axis_index("c")
        s = lax.axis_index("s")
        g = c * NS + s

        q_sz = rpt // 5
        def zrow(i, _):
            for j in range(H // LANES):
                stage[i, pl.ds(j * LANES, LANES)] = jnp.zeros((LANES,), F32)
            return 0
        lax.fori_loop(0, q_sz, zrow, 0)
        def zq(q, _):
            pltpu.sync_copy(stage, aggsh.at[pl.ds(s * rpt + q * q_sz, q_sz)])
            return 0
        lax.fori_loop(0, 5, zq, 0)

        pltpu.sync_copy(src_hbm.at[g], srcv)
        pltpu.sync_copy(dst_hbm.at[g], dstv)
        plsc.subcore_barrier()

        def chunk(kk, _):
            cp1 = pltpu.async_copy(a_hbm.at[srcv.at[kk]], av, sem1)
            cp2 = pltpu.async_copy(b_hbm.at[dstv.at[kk]], bv, sem2)
            cp1.wait()
            cp2.wait()

            def ebody(i, _):
                for j in range(H // LANES):
                    sl = pl.ds(j * LANES, LANES)
                    x = av[i, sl] + bv[i, sl]
                    mv[i, sl] = _silu(x)
                return 0
            lax.fori_loop(0, c_sz, ebody, 0)
            pltpu.sync_copy(mv, aggsh.at[dstv.at[kk]], add=True)
            return 0
        lax.fori_loop(0, nch, chunk, 0)

        plsc.subcore_barrier()

        def cq(q, _):
            pltpu.sync_copy(aggsh.at[pl.ds(s * rpt + q * q_sz, q_sz)], stage)

            @pl.when(c == 0)
            def _():
                pltpu.sync_copy(stage, out0.at[s, pl.ds(q * q_sz, q_sz)])

            @pl.when(c == 1)
            def _():
                pltpu.sync_copy(stage, out1.at[s, pl.ds(q * q_sz, q_sz)])
            return 0
        lax.fori_loop(0, 5, cq, 0)

    return k(a_tab, b_tab, src3d, dst3d)


def _sc_edge_final(p_tab, q_tab, src3d, dst3d):
    _, nch, c_sz = src3d.shape
    ept = nch * c_sz
    e = NC * NS * ept

    mesh = plsc.VectorSubcoreMesh(core_axis_name="c", subcore_axis_name="s")

    @functools.partial(
        pl.kernel,
        out_type=jax.ShapeDtypeStruct((e, H), F32),
        mesh=mesh,
        compiler_params=pltpu.CompilerParams(use_tc_tiling_on_sc=False),
        scratch_types=[
            pltpu.VMEM((nch, c_sz), jnp.int32),
            pltpu.VMEM((nch, c_sz), jnp.int32),
            pltpu.VMEM((c_sz, H), F32),
            pltpu.VMEM((c_sz, H), F32),
            pltpu.VMEM((c_sz, H), F32),
            pltpu.SemaphoreType.DMA,
            pltpu.SemaphoreType.DMA,
        ],
    )
    def k(p_hbm, q_hbm, src_hbm, dst_hbm, uout,
          srcv, dstv, av, bv, uv, sem1, sem2):
        c = lax.axis_index("c")
        s = lax.axis_index("s")
        g = c * NS + s

        pltpu.sync_copy(src_hbm.at[g], srcv)
        pltpu.sync_copy(dst_hbm.at[g], dstv)

        def chunk(kk, _):
            cp1 = pltpu.async_copy(p_hbm.at[srcv.at[kk]], av, sem1)
            cp2 = pltpu.async_copy(q_hbm.at[dstv.at[kk]], bv, sem2)
            cp1.wait()
            cp2.wait()

            def ebody(i, _):
                for j in range(H // LANES):
                    sl = pl.ds(j * LANES, LANES)
                    x = av[i, sl] + bv[i, sl]
                    uv[i, sl] = _silu(x)
                return 0
            lax.fori_loop(0, c_sz, ebody, 0)
            pltpu.sync_copy(uv, uout.at[pl.ds(g * ept + kk * c_sz, c_sz)])
            return 0
        lax.fori_loop(0, nch, chunk, 0)

    return k(p_tab, q_tab, src3d, dst3d)


def _tc_rate(u, w2, b2, block_rows=8000):
    e = u.shape[0]

    def body(u_ref, w2_ref, b2_ref, r_ref):
        z = jnp.dot(u_ref[...], w2_ref[...], preferred_element_type=F32)
        r_ref[...] = jax.nn.softplus(z + b2_ref[...])

    return pl.pallas_call(
        body,
        grid=(e // block_rows,),
        in_specs=[
            pl.BlockSpec((block_rows, H), lambda i: (i, 0)),
            pl.BlockSpec((H, 1), lambda i: (0, 0)),
            pl.BlockSpec((1, 1), lambda i: (0, 0)),
        ],
        out_specs=pl.BlockSpec((block_rows, 1), lambda i: (i, 0)),
        out_shape=jax.ShapeDtypeStruct((e, 1), F32),
    )(u, w2, b2.reshape(1, 1))


def _sc_matrix(src4d, dst4d, r4d, n):
    _, nhalf, half, c_sz = src4d.shape
    nn = n * n
    zpt = nn // (NC * NS)
    zbuf = 25000
    nzcp = zpt // zbuf

    ZU = 5
    mesh = plsc.VectorSubcoreMesh(core_axis_name="c", subcore_axis_name="s")

    @functools.partial(
        pl.kernel,
        out_type=jax.ShapeDtypeStruct((nn,), F32),
        mesh=mesh,
        compiler_params=pltpu.CompilerParams(use_tc_tiling_on_sc=False),
        scratch_types=[
            pltpu.VMEM((half, c_sz), jnp.int32),
            pltpu.VMEM((half, c_sz), jnp.int32),
            pltpu.VMEM((half, c_sz), F32),
            pltpu.VMEM((2, c_sz), jnp.int32),
            pltpu.VMEM((zbuf,), F32),
            pltpu.SemaphoreType.DMA,
            pltpu.SemaphoreType.DMA,
            pltpu.SemaphoreType.DMA,
            pltpu.SemaphoreType.DMA,
            pltpu.SemaphoreType.DMA,
        ],
    )
    def k(src_hbm, dst_hbm, r_hbm, m_out, srcv, dstv, rv, idxv, zv,
          z0, z1, z2, z3, z4):
        c = lax.axis_index("c")
        s = lax.axis_index("s")
        g = c * NS + s

        def zfill(i, _):
            zv[pl.ds(i * LANES, LANES)] = jnp.zeros((LANES,), F32)
            return 0
        lax.fori_loop(0, zbuf // LANES, zfill, 0)
        if zbuf % LANES:
            zv[pl.ds(zbuf - LANES, LANES)] = jnp.zeros((LANES,), F32)

        zsems = (z0, z1, z2, z3, z4)

        def zout(i, _):
            base = g * zpt + i * (ZU * zbuf)
            cps = []
            for u in range(ZU):
                off = pl.multiple_of(base + u * zbuf, 8)
                cps.append(pltpu.async_copy(
                    zv, m_out.at[pl.ds(off, zbuf)], zsems[u]))
            for cp in cps:
                cp.wait()
            return 0
        lax.fori_loop(0, nzcp // ZU, zout, 0)
        plsc.subcore_barrier()

        for h in range(nhalf):
            pltpu.sync_copy(src_hbm.at[s, h], srcv)
            pltpu.sync_copy(dst_hbm.at[s, h], dstv)
            pltpu.sync_copy(r_hbm.at[s, h], rv)

            def chunk2(kk, _):
                def ib0(j, _):
                    sl = pl.ds(j * LANES, LANES)
                    idxv[0, sl] = srcv[2 * kk, sl] * n + dstv[2 * kk, sl]
                    return 0
                lax.fori_loop(0, c_sz // LANES, ib0, 0)
                cp0 = pltpu.async_copy(rv.at[2 * kk], m_out.at[idxv.at[0]], z0)

                def ib1(j, _):
                    sl = pl.ds(j * LANES, LANES)
                    idxv[1, sl] = (srcv[2 * kk + 1, sl] * n
                                   + dstv[2 * kk + 1, sl])
                    return 0
                lax.fori_loop(0, c_sz // LANES, ib1, 0)
                cp1 = pltpu.async_copy(rv.at[2 * kk + 1],
                                       m_out.at[idxv.at[1]], z1)
                cp0.wait()
                cp1.wait()
                return 0
            lax.fori_loop(0, half // 2, chunk2, 0)

    return k(src4d, dst4d, r4d)


def _tc_diag(m, block_rows=128):
    n = m.shape[0]

    def body(m_ref, tile_ref, out_ref):
        rowsum = jnp.sum(m_ref[...], axis=1)
        rr = lax.broadcasted_iota(jnp.int32, (block_rows, block_rows), 0)
        cc = lax.broadcasted_iota(jnp.int32, (block_rows, block_rows), 1)
        out_ref[...] = jnp.where(rr == cc, -rowsum[:, None], tile_ref[...])

    return pl.pallas_call(
        body,
        grid=(pl.cdiv(n, block_rows),),
        in_specs=[
            pl.BlockSpec((block_rows, n), lambda i: (i, 0)),
            pl.BlockSpec((block_rows, block_rows), lambda i: (i, i)),
        ],
        out_specs=pl.BlockSpec((block_rows, block_rows), lambda i: (i, i)),
        out_shape=jax.ShapeDtypeStruct((n, n), F32),
        input_output_aliases={0: 0},
    )(m, m)


def kernel(mu, t, context, edge_index, mp_params, edge_mlp_params):
    n = mu.shape[0]
    e = edge_index.shape[1]
    c_sz = 160

    src = edge_index[0].astype(jnp.int32)
    dst = edge_index[1].astype(jnp.int32)
    nw = NC * NS
    nch = e // (nw * c_sz)
    src3d = src.reshape(nw, nch, c_sz)
    dst3d = dst.reshape(nw, nch, c_sz)
    half9 = e // (NS * 2 * c_sz)
    src4d = src.reshape(NS, 2, half9, c_sz)
    dst4d = dst.reshape(NS, 2, half9, c_sz)

    ctx = context.astype(F32)
    d0 = 2 + ctx.shape[1]
    h0 = jnp.concatenate(
        [mu.reshape(n, 1), jnp.broadcast_to(t.reshape(1, 1), (n, 1)), ctx,
         jnp.zeros((n, H - d0), F32)], axis=1)

    def pad_rows(w):
        return jnp.concatenate(
            [w, jnp.zeros((H - w.shape[0], w.shape[1]), F32)], axis=0)

    w1, b1, w2, b2 = edge_mlp_params
    msg_w, msg_b, upd_wt, upd_wb, upd_b = [], [], [], [], []
    for (wm, bm, wu, bu) in mp_params:
        d = wm.shape[0] // 2
        msg_w.append(jnp.concatenate(
            [pad_rows(wm[:d]), pad_rows(wm[d:])], axis=1))
        msg_b.append(jnp.concatenate([bm, jnp.zeros((H,), F32)]))
        upd_wt.append(pad_rows(wu[:d]))
        upd_wb.append(wu[d:])
        upd_b.append(bu)
    msg_w.append(jnp.concatenate([w1[:H], w1[H:]], axis=1))
    msg_b.append(jnp.concatenate([b1, jnp.zeros((H,), F32)]))

    a_tab, b_tab = _tc_ab(h0, msg_w[0], msg_b[0])
    h = h0
    for l in range(len(mp_params)):
        agg0, agg1 = _sc_edge(a_tab, b_tab, src3d, dst3d)
        agg0 = agg0.reshape(n, H)
        agg1 = agg1.reshape(n, H)
        h, a_tab, b_tab = _tc_update(
            h, agg0, agg1, upd_wt[l], upd_wb[l], upd_b[l],
            msg_w[l + 1], msg_b[l + 1])

    u = _sc_edge_final(a_tab, b_tab, src3d, dst3d)
    r = _tc_rate(u, w2, b2).reshape(e)
    m_flat = _sc_matrix(src4d, dst4d, r.reshape(NS, 2, half9, c_sz), n)
    return _tc_diag(m_flat.reshape(n, n))
